# baseline (speedup 1.0000x reference)
import functools

import jax
import jax.numpy as jnp
from jax import lax
from jax.experimental import pallas as pl
from jax.experimental.pallas import tpu as pltpu
from jax.experimental.pallas import tpu_sc as plsc

N_NODES = 10000
N_EDGES = 320000
D = 128
HIDDEN = 256

NC = 2
NS = 16
NW = NC * NS
CHUNK = 128
N_CHUNKS = N_EDGES // CHUNK
CHUNKS_PER_W = 80
LAST_W = NW - 1
LAST_REAL = N_CHUNKS - LAST_W * CHUNKS_PER_W
N_PAD = CHUNKS_PER_W - LAST_REAL
N_SUPER = CHUNKS_PER_W
AGG_PAD = 10240
ZERO_PER_TILE = AGG_PAD // NS
DEAD_ROW = N_NODES + 64
EXP_TILES = 10


def _sc_segment_sum(edge_feats, ei3, pad_idx):
    mesh = plsc.VectorSubcoreMesh(
        core_axis_name="c", subcore_axis_name="s",
        num_cores=NC, num_subcores=NS)

    @functools.partial(
        pl.kernel,
        out_type=jax.ShapeDtypeStruct((NC, N_NODES, D), jnp.float32),
        mesh=mesh,
        scratch_types=[
            pltpu.VMEM((CHUNKS_PER_W, CHUNK), jnp.int32),
            pltpu.VMEM((CHUNK, D), jnp.float32),
            pltpu.VMEM((CHUNK, D), jnp.float32),
            pltpu.VMEM_SHARED((AGG_PAD, D), jnp.float32),
            pltpu.SemaphoreType.DMA,
            pltpu.SemaphoreType.DMA,
        ],
    )
    def seg_sum(edge_hbm, ei_hbm, pad_hbm, out_hbm, idx_v,
                buf_a, buf_b, agg_sh, sem_a, sem_b):
        cid = lax.axis_index("c")
        sid = lax.axis_index("s")
        wid = sid * NC + cid
        c0 = wid * CHUNKS_PER_W

        @pl.when(wid < LAST_W)
        def _():
            pltpu.make_async_copy(
                ei_hbm.at[1, pl.ds(c0, CHUNKS_PER_W)], idx_v, sem_b).start()

        @pl.when(wid == LAST_W)
        def _():
            pltpu.make_async_copy(
                ei_hbm.at[1, pl.ds(LAST_W * CHUNKS_PER_W, LAST_REAL)],
                idx_v.at[pl.ds(0, LAST_REAL)], sem_b).start()
            pltpu.make_async_copy(
                pad_hbm, idx_v.at[pl.ds(LAST_REAL, N_PAD)], sem_b).start()

        def zrow(r, _):
            for k in range(D // 16):
                buf_a[r, pl.ds(k * 16, 16)] = jnp.zeros((16,), jnp.float32)
            return 0
        lax.fori_loop(0, CHUNK, zrow, 0)
        for z in range(ZERO_PER_TILE // CHUNK):
            pltpu.sync_copy(
                buf_a, agg_sh.at[pl.ds(sid * ZERO_PER_TILE + z * CHUNK, CHUNK)])

        pltpu.make_async_copy(
            ei_hbm.at[1, pl.ds(0, CHUNKS_PER_W)], idx_v, sem_b).wait()
        plsc.subcore_barrier()

        bufs = (buf_a, buf_b)
        sems = (sem_a, sem_b)

        def load(s, b):
            cc = c0 + s
            src_c = lax.select(cc < N_CHUNKS, cc, cc - N_CHUNKS)
            return pltpu.make_async_copy(
                edge_hbm.at[pl.ds(src_c * CHUNK, CHUNK)], bufs[b], sems[b])

        load(0, 0).start()
        load(1, 1).start()

        def outer(t, _):
            for b in range(2):
                s = 2 * t + b
                load(s, b).wait()
                pltpu.sync_copy(bufs[b], agg_sh.at[idx_v.at[s]], add=True)

                @pl.when(s + 2 < N_SUPER)
                def _():
                    load(s + 2, b).start()
            return 0
        lax.fori_loop(0, N_SUPER // 2, outer, 0)
        plsc.subcore_barrier()

        @pl.when(sid < EXP_TILES)
        def _():
            rows = N_NODES // EXP_TILES
            pltpu.sync_copy(agg_sh.at[pl.ds(sid * rows, rows)],
                            out_hbm.at[cid, pl.ds(sid * rows, rows)])

    return seg_sum(edge_feats, ei3, pad_idx)


COPY_BLK = 20000


def _copy_body(src_ref, dst_ref):
    dst_ref[...] = src_ref[...]


def _tc_edge_copy(edge_feats):
    return pl.pallas_call(
        _copy_body,
        grid=(N_EDGES // COPY_BLK,),
        in_specs=[pl.BlockSpec((COPY_BLK, D), lambda i: (i, 0))],
        out_specs=pl.BlockSpec((COPY_BLK, D), lambda i: (i, 0)),
        out_shape=jax.ShapeDtypeStruct((N_EDGES, D), jnp.float32),
    )(edge_feats)


def _tc_mlp_body(parts_0, parts_1, nf_ref, w1a_ref, w1b_ref, b1_ref,
                 w2_ref, b2_ref, g_ref, b_ref, out_ref):
    agg = parts_0[0] + parts_1[0]
    nf = nf_ref[...]
    h = (jnp.dot(agg, w1a_ref[...], preferred_element_type=jnp.float32)
         + jnp.dot(nf, w1b_ref[...], preferred_element_type=jnp.float32)
         + b1_ref[...])
    h = h * jax.nn.sigmoid(h)
    h = jnp.dot(h, w2_ref[...], preferred_element_type=jnp.float32) + b2_ref[...]
    mu = jnp.mean(h, axis=-1, keepdims=True)
    hc = h - mu
    var = jnp.mean(hc * hc, axis=-1, keepdims=True)
    h = hc * lax.rsqrt(var + 1e-5) * g_ref[...] + b_ref[...]
    out_ref[...] = h + nf


NODE_BLK = 10000


def _tc_mlp(partials, node_feats, W1, b1, W2, b2, ln_g, ln_b):
    w1a = W1[:D]
    w1b = W1[D:]
    fixed = lambda i: (0, 0)
    return pl.pallas_call(
        _tc_mlp_body,
        grid=(N_NODES // NODE_BLK,),
        in_specs=[
            pl.BlockSpec((1, NODE_BLK, D), lambda i: (0, i, 0)),
            pl.BlockSpec((1, NODE_BLK, D), lambda i: (1, i, 0)),
            pl.BlockSpec((NODE_BLK, D), lambda i: (i, 0)),
            pl.BlockSpec((D, HIDDEN), fixed),
            pl.BlockSpec((D, HIDDEN), fixed),
            pl.BlockSpec((1, HIDDEN), fixed),
            pl.BlockSpec((HIDDEN, D), fixed),
            pl.BlockSpec((1, D), fixed),
            pl.BlockSpec((1, D), fixed),
            pl.BlockSpec((1, D), fixed),
        ],
        out_specs=pl.BlockSpec((NODE_BLK, D), lambda i: (i, 0)),
        out_shape=jax.ShapeDtypeStruct((N_NODES, D), jnp.float32),
    )(partials, partials, node_feats, w1a, w1b, b1.reshape(1, HIDDEN),
      W2, b2.reshape(1, D), ln_g.reshape(1, D), ln_b.reshape(1, D))


def kernel(edge_feats, node_feats, edge_index, W1, b1, W2, b2, ln_g, ln_b):
    ei3 = edge_index.astype(jnp.int32).reshape(2, N_CHUNKS, CHUNK)
    pad_idx = jnp.full((N_PAD, CHUNK), DEAD_ROW, jnp.int32)
    edge_out = _tc_edge_copy(edge_feats)
    partials = _sc_segment_sum(edge_feats, ei3, pad_idx)
    node_new = _tc_mlp(partials, node_feats, W1, b1, W2, b2, ln_g, ln_b)
    return (edge_out, node_new)


# speedup vs baseline: 1.0071x; 1.0071x over previous
import functools

import jax
import jax.numpy as jnp
from jax import lax
from jax.experimental import pallas as pl
from jax.experimental.pallas import tpu as pltpu
from jax.experimental.pallas import tpu_sc as plsc

N_NODES = 10000
N_EDGES = 320000
D = 128
HIDDEN = 256

NC = 2
NS = 16
NW = NC * NS
CHUNK = 128
N_CHUNKS = N_EDGES // CHUNK
CHUNKS_PER_W = 80
LAST_W = NW - 1
LAST_REAL = N_CHUNKS - LAST_W * CHUNKS_PER_W
N_PAD = CHUNKS_PER_W - LAST_REAL
N_SUPER = CHUNKS_PER_W
AGG_PAD = 10240
ZERO_PER_TILE = AGG_PAD // NS
DEAD_ROW = N_NODES + 64
EXP_TILES = 10


def _sc_segment_sum(edge_feats, ei3, pad_idx):
    mesh = plsc.VectorSubcoreMesh(
        core_axis_name="c", subcore_axis_name="s",
        num_cores=NC, num_subcores=NS)

    @functools.partial(
        pl.kernel,
        out_type=jax.ShapeDtypeStruct((NC, N_NODES, D), jnp.float32),
        mesh=mesh,
        scratch_types=[
            pltpu.VMEM((CHUNKS_PER_W, CHUNK), jnp.int32),
            pltpu.VMEM((CHUNK, D), jnp.float32),
            pltpu.VMEM((CHUNK, D), jnp.float32),
            pltpu.VMEM_SHARED((AGG_PAD, D), jnp.float32),
            pltpu.SemaphoreType.DMA,
            pltpu.SemaphoreType.DMA,
        ],
    )
    def seg_sum(edge_hbm, ei_hbm, pad_hbm, out_hbm, idx_v,
                buf_a, buf_b, agg_sh, sem_a, sem_b):
        cid = lax.axis_index("c")
        sid = lax.axis_index("s")
        wid = sid * NC + cid
        c0 = wid * CHUNKS_PER_W

        @pl.when(wid < LAST_W)
        def _():
            pltpu.make_async_copy(
                ei_hbm.at[1, pl.ds(c0, CHUNKS_PER_W)], idx_v, sem_b).start()

        @pl.when(wid == LAST_W)
        def _():
            pltpu.make_async_copy(
                ei_hbm.at[1, pl.ds(LAST_W * CHUNKS_PER_W, LAST_REAL)],
                idx_v.at[pl.ds(0, LAST_REAL)], sem_b).start()
            pltpu.make_async_copy(
                pad_hbm, idx_v.at[pl.ds(LAST_REAL, N_PAD)], sem_b).start()

        def zrow(r, _):
            for k in range(D // 16):
                buf_a[r, pl.ds(k * 16, 16)] = jnp.zeros((16,), jnp.float32)
            return 0
        lax.fori_loop(0, CHUNK, zrow, 0)
        for z in range(ZERO_PER_TILE // CHUNK):
            pltpu.sync_copy(
                buf_a, agg_sh.at[pl.ds(sid * ZERO_PER_TILE + z * CHUNK, CHUNK)])

        pltpu.make_async_copy(
            ei_hbm.at[1, pl.ds(0, CHUNKS_PER_W)], idx_v, sem_b).wait()
        plsc.subcore_barrier()

        bufs = (buf_a, buf_b)
        sems = (sem_a, sem_b)

        def load(s, b):
            cc = c0 + s
            src_c = lax.select(cc < N_CHUNKS, cc, cc - N_CHUNKS)
            return pltpu.make_async_copy(
                edge_hbm.at[pl.ds(src_c * CHUNK, CHUNK)], bufs[b], sems[b])

        load(0, 0).start()
        load(1, 1).start()

        def outer(t, _):
            for b in range(2):
                s = 2 * t + b
                load(s, b).wait()
                pltpu.sync_copy(bufs[b], agg_sh.at[idx_v.at[s]], add=True)

                @pl.when(s + 2 < N_SUPER)
                def _():
                    load(s + 2, b).start()
            return 0
        lax.fori_loop(0, N_SUPER // 2, outer, 0)
        plsc.subcore_barrier()

        @pl.when(sid < EXP_TILES)
        def _():
            rows = N_NODES // EXP_TILES
            pltpu.sync_copy(agg_sh.at[pl.ds(sid * rows, rows)],
                            out_hbm.at[cid, pl.ds(sid * rows, rows)])

    return seg_sum(edge_feats, ei3, pad_idx)


COPY_BLK = 20000


def _copy_body(src_ref, dst_ref):
    dst_ref[...] = src_ref[...]


def _tc_edge_copy(edge_feats):
    return pl.pallas_call(
        _copy_body,
        grid=(N_EDGES // COPY_BLK,),
        in_specs=[pl.BlockSpec((COPY_BLK, D), lambda i: (i, 0))],
        out_specs=pl.BlockSpec((COPY_BLK, D), lambda i: (i, 0)),
        out_shape=jax.ShapeDtypeStruct((N_EDGES, D), jnp.float32),
    )(edge_feats)


def _tc_mlp_body(parts_0, parts_1, nf_ref, w1a_ref, w1b_ref, b1_ref,
                 w2_ref, b2_ref, g_ref, b_ref, out_ref):
    agg = parts_0[0] + parts_1[0]
    nf = nf_ref[...]
    h = (jnp.dot(agg, w1a_ref[...], preferred_element_type=jnp.float32)
         + jnp.dot(nf, w1b_ref[...], preferred_element_type=jnp.float32)
         + b1_ref[...])
    h = h * jax.nn.sigmoid(h)
    h = jnp.dot(h, w2_ref[...], preferred_element_type=jnp.float32) + b2_ref[...]
    mu = jnp.mean(h, axis=-1, keepdims=True)
    hc = h - mu
    var = jnp.mean(hc * hc, axis=-1, keepdims=True)
    h = hc * lax.rsqrt(var + 1e-5) * g_ref[...] + b_ref[...]
    out_ref[...] = h + nf


NODE_BLK = 2000


def _tc_mlp(partials, node_feats, W1, b1, W2, b2, ln_g, ln_b):
    w1a = W1[:D]
    w1b = W1[D:]
    fixed = lambda i: (0, 0)
    return pl.pallas_call(
        _tc_mlp_body,
        grid=(N_NODES // NODE_BLK,),
        in_specs=[
            pl.BlockSpec((1, NODE_BLK, D), lambda i: (0, i, 0)),
            pl.BlockSpec((1, NODE_BLK, D), lambda i: (1, i, 0)),
            pl.BlockSpec((NODE_BLK, D), lambda i: (i, 0)),
            pl.BlockSpec((D, HIDDEN), fixed),
            pl.BlockSpec((D, HIDDEN), fixed),
            pl.BlockSpec((1, HIDDEN), fixed),
            pl.BlockSpec((HIDDEN, D), fixed),
            pl.BlockSpec((1, D), fixed),
            pl.BlockSpec((1, D), fixed),
            pl.BlockSpec((1, D), fixed),
        ],
        out_specs=pl.BlockSpec((NODE_BLK, D), lambda i: (i, 0)),
        out_shape=jax.ShapeDtypeStruct((N_NODES, D), jnp.float32),
    )(partials, partials, node_feats, w1a, w1b, b1.reshape(1, HIDDEN),
      W2, b2.reshape(1, D), ln_g.reshape(1, D), ln_b.reshape(1, D))


def kernel(edge_feats, node_feats, edge_index, W1, b1, W2, b2, ln_g, ln_b):
    ei3 = edge_index.astype(jnp.int32).reshape(2, N_CHUNKS, CHUNK)
    pad_idx = jnp.full((N_PAD, CHUNK), DEAD_ROW, jnp.int32)
    edge_out = _tc_edge_copy(edge_feats)
    partials = _sc_segment_sum(edge_feats, ei3, pad_idx)
    node_new = _tc_mlp(partials, node_feats, W1, b1, W2, b2, ln_g, ln_b)
    return (edge_out, node_new)
